# baseline (speedup 1.0000x reference)
import functools

import jax
import jax.numpy as jnp
from jax import lax
from jax.experimental import pallas as pl
from jax.experimental.pallas import tpu as pltpu
from jax.experimental.pallas import tpu_sc as plsc

_MIN_W = 1e-3
_MIN_H = 1e-3
_NB = 32
_NW = 32
_CHUNK = 64


def _softmax1(x):
    m = jnp.max(x, axis=-1, keepdims=True)
    e = jnp.exp(x - m)
    return e / jnp.sum(e, axis=-1, keepdims=True)


def _prep_body(uw_ref, uh_ref, udl_ref, udr_ref,
               cw_ref, a_ref, b_ref, c_ref, d_ref, g_ref):
    nb = _NB
    uw = uw_ref[...]
    uh = uh_ref[...]
    w = _MIN_W + (1.0 - _MIN_W * nb) * _softmax1(uw)
    h = _MIN_H + (1.0 - _MIN_H * nb) * _softmax1(uh)

    row = lax.broadcasted_iota(jnp.int32, (nb, nb), 0)
    col = lax.broadcasted_iota(jnp.int32, (nb, nb), 1)
    tri = (row <= col).astype(jnp.float32)
    cums_w = jnp.dot(w, tri, preferred_element_type=jnp.float32)
    cums_h = jnp.dot(h, tri, preferred_element_type=jnp.float32)

    zero = jnp.zeros_like(w[:, 0:1])
    cw = jnp.concatenate([zero, cums_w[:, : nb - 1]], axis=1)
    dtab = jnp.concatenate([zero, cums_h[:, : nb - 1]], axis=1)

    s = h / w
    s_lo, s_hi = s[:, : nb - 1], s[:, 1:]
    w_lo, w_hi = w[:, : nb - 1], w[:, 1:]
    min1 = jnp.minimum(jnp.abs(s_lo), jnp.abs(s_hi))
    min2 = 0.5 * (w_hi * s_lo + w_lo * s_hi) / (w_lo + w_hi)
    dmid = jnp.minimum(min1, min2) * (jnp.sign(s_lo) + jnp.sign(s_hi))

    d0 = jax.nn.sigmoid(udl_ref[...]) * 3.0 * s[:, 0:1]
    dN = jax.nn.sigmoid(udr_ref[...]) * 3.0 * s[:, nb - 1 : nb]
    dlo = jnp.concatenate([d0, dmid], axis=1)
    dhi = jnp.concatenate([dmid, dN], axis=1)

    cw_ref[...] = jnp.concatenate(
        [cw, jnp.full((cw.shape[0], 1), 2.0, jnp.float32)], axis=1)
    pad = jnp.zeros((cw.shape[0], 1), jnp.float32)
    a_ref[...] = jnp.concatenate([(dlo + dhi - 2.0 * s) / (w * w), pad], 1)
    b_ref[...] = jnp.concatenate([(3.0 * s - 2.0 * dlo - dhi) / w, pad], 1)
    c_ref[...] = jnp.concatenate([dlo, pad], 1)
    d_ref[...] = jnp.concatenate([dtab, pad], 1)

    lane = lax.broadcasted_iota(jnp.int32, (1, 128), 1)
    word = jnp.zeros((cw.shape[0], 128), jnp.int32)
    for p in range(4):
        xg = (4 * lane + p).astype(jnp.float32) * (1.0 / 512.0)
        gp = jnp.zeros((cw.shape[0], 128), jnp.int32)
        for k in range(1, nb):
            gp = gp + jnp.where(xg >= cw[:, k : k + 1], 1, 0)
        word = jnp.bitwise_or(word, jnp.left_shift(gp, 8 * p))
    g_ref[...] = jnp.concatenate(
        [word, jnp.zeros((cw.shape[0], 1), jnp.int32)], 1)


def _prep_tables(uw, uh, udl, udr):
    D, nb = uw.shape
    shape = jax.ShapeDtypeStruct((D, nb), jnp.float32)
    return pl.pallas_call(
        _prep_body,
        out_shape=[jax.ShapeDtypeStruct((D, nb + 1), jnp.float32)] * 5
        + [jax.ShapeDtypeStruct((D, 129), jnp.int32)],
    )(uw, uh, udl, udr)


def _log_abs(y):
    y = jnp.abs(y)
    i = plsc.bitcast(y, jnp.int32)
    e = jnp.right_shift(i, 23) - 127
    m = plsc.bitcast(
        jnp.bitwise_or(jnp.bitwise_and(i, 0x7FFFFF), 0x3F800000), jnp.float32)
    big = m >= 1.4142135381698608
    m = jnp.where(big, m * 0.5, m)
    ef = e.astype(jnp.float32) + jnp.where(big, 1.0, 0.0)
    z = (m - 1.0) / (m + 1.0)
    z2 = z * z
    t = z2 * (1.0 / 7.0) + (1.0 / 5.0)
    t = z2 * t + (1.0 / 3.0)
    t = z2 * t + 1.0
    return ef * 0.6931471805599453 + 2.0 * z * t


_T = 256 * (_NB + 1)
_OFF_A, _OFF_B, _OFF_C, _OFF_D, _OFF_G = _T, 2 * _T, 3 * _T, 4 * _T, 5 * _T


def _sc_body(n_rows, x_hbm, tab_h, out_hbm, lad_hbm,
             tab_v, xin, xout, lad_v):
    wid = lax.axis_index("s") * 2 + lax.axis_index("c")
    pltpu.sync_copy(tab_h, tab_v)

    rows_per_w = n_rows // _NW
    n_chunks = rows_per_w // _CHUNK
    iota = lax.broadcasted_iota(jnp.int32, (16,), 0)

    def chunk_body(ch, carry):
        row0 = wid * rows_per_w + ch * _CHUNK
        pltpu.sync_copy(x_hbm.at[pl.ds(row0 * 256, _CHUNK * 256)], xin)

        @plsc.parallel_loop(0, _CHUNK, unroll=2)
        def row_body(r):
            acc_m = jnp.ones((16,), jnp.float32)
            acc_e = jnp.zeros((16,), jnp.int32)
            xs = [xin[pl.ds(r * 256 + j * 16, 16)] for j in range(16)]
            outs = []
            for j in range(16):
                x = xs[j]
                dvec = iota + (j * 16)
                base33 = dvec * (_NB + 1)
                c512 = (x * 512.0).astype(jnp.int32)
                gw = plsc.bitcast(plsc.load_gather(
                    tab_v,
                    [dvec * 129 + jnp.right_shift(c512, 2) + _OFF_G]),
                    jnp.int32)
                sh_amt = jnp.left_shift(jnp.bitwise_and(c512, 3), 3)
                g = jnp.bitwise_and(jnp.right_shift(gw, sh_amt), 0xFF)
                v0 = plsc.load_gather(tab_v, [base33 + g])
                v1 = plsc.load_gather(tab_v, [base33 + g + 1])
                v2 = plsc.load_gather(
                    tab_v, [base33 + jnp.minimum(g + 2, _NB)])
                m1 = x >= v1
                m2 = x >= v2
                binx = (g + jnp.where(m1, 1, 0) + jnp.where(m2, 1, 0))
                wv = jnp.where(m1, jnp.where(m2, v2, v1), v0)
                flat = base33 + binx
                av = plsc.load_gather(tab_v, [flat + _OFF_A])
                bv = plsc.load_gather(tab_v, [flat + _OFF_B])
                cv = plsc.load_gather(tab_v, [flat + _OFF_C])
                dv = plsc.load_gather(tab_v, [flat + _OFF_D])
                sh = x - wv
                s2 = sh * sh
                s3 = s2 * sh
                p = av * s3 + bv * s2 + cv * sh + dv
                outs.append(jnp.clip(p, 0.0, 1.0))
                deriv = 3.0 * av * s2 + 2.0 * bv * sh + cv
                db = plsc.bitcast(jnp.abs(deriv), jnp.int32)
                acc_e = acc_e + jnp.right_shift(db, 23)
                acc_m = acc_m * plsc.bitcast(
                    jnp.bitwise_or(jnp.bitwise_and(db, 0x7FFFFF), 0x3F800000),
                    jnp.float32)
            for j in range(16):
                xout[pl.ds(r * 256 + j * 16, 16)] = outs[j]
            lad16 = _log_abs(acc_m) + (
                acc_e - 16 * 127).astype(jnp.float32) * 0.6931471805599453
            tot = jnp.broadcast_to(jnp.sum(lad16), (16,))
            plsc.store_scatter(lad_v, [jnp.broadcast_to(r, (16,))], tot,
                               mask=iota == 0)
        pltpu.sync_copy(xout, out_hbm.at[pl.ds(row0 * 256, _CHUNK * 256)])
        pltpu.sync_copy(lad_v, lad_hbm.at[pl.ds(row0, _CHUNK)])
        return carry

    lax.fori_loop(0, n_chunks, chunk_body, 0)


def _sc_main(x_flat, tab, n_rows):
    B = n_rows
    mesh = plsc.VectorSubcoreMesh(
        core_axis_name="c", subcore_axis_name="s", num_cores=2,
        num_subcores=16)
    f = functools.partial(
        pl.kernel,
        out_type=[
            jax.ShapeDtypeStruct((B * 256,), jnp.float32),
            jax.ShapeDtypeStruct((B,), jnp.float32),
        ],
        mesh=mesh,
        scratch_types=[
            pltpu.VMEM((5 * _T + 256 * 129,), jnp.float32),
            pltpu.VMEM((_CHUNK * 256,), jnp.float32),
            pltpu.VMEM((_CHUNK * 256,), jnp.float32),
            pltpu.VMEM((_CHUNK,), jnp.float32),
        ],
        compiler_params=pltpu.CompilerParams(needs_layout_passes=False),
    )(functools.partial(_sc_body, n_rows))
    return f(x_flat, tab)


def _softmax0(x):
    m = jnp.max(x, axis=0, keepdims=True)
    e = jnp.exp(x - m)
    return e / jnp.sum(e, axis=0, keepdims=True)


def _tc_tables(uwt, uht, udlt, udrt):
    nb = _NB
    w = _MIN_W + (1.0 - _MIN_W * nb) * _softmax0(uwt)
    h = _MIN_H + (1.0 - _MIN_H * nb) * _softmax0(uht)

    row = lax.broadcasted_iota(jnp.int32, (nb, nb), 0)
    col = lax.broadcasted_iota(jnp.int32, (nb, nb), 1)
    tri = (col <= row).astype(jnp.float32)
    cums_w = jnp.dot(tri, w, preferred_element_type=jnp.float32)
    cums_h = jnp.dot(tri, h, preferred_element_type=jnp.float32)

    zero = jnp.zeros_like(w[0:1])
    cw = jnp.concatenate([zero, cums_w[: nb - 1]], axis=0)
    dtab = jnp.concatenate([zero, cums_h[: nb - 1]], axis=0)

    s = h / w
    s_lo, s_hi = s[: nb - 1], s[1:]
    w_lo, w_hi = w[: nb - 1], w[1:]
    min1 = jnp.minimum(jnp.abs(s_lo), jnp.abs(s_hi))
    min2 = 0.5 * (w_hi * s_lo + w_lo * s_hi) / (w_lo + w_hi)
    dmid = jnp.minimum(min1, min2) * (jnp.sign(s_lo) + jnp.sign(s_hi))

    d0 = jax.nn.sigmoid(udlt) * 3.0 * s[0:1]
    dN = jax.nn.sigmoid(udrt) * 3.0 * s[nb - 1 : nb]
    dlo = jnp.concatenate([d0, dmid], axis=0)
    dhi = jnp.concatenate([dmid, dN], axis=0)

    a = (dlo + dhi - 2.0 * s) / (w * w)
    b = (3.0 * s - 2.0 * dlo - dhi) / w
    return cw, a, b, dlo, dtab


def _tc_body(x_ref, uwt_ref, uht_ref, udlt_ref, udrt_ref, out_ref, lad_ref):
    cw, a, b, c, d = _tc_tables(
        uwt_ref[...], uht_ref[...], udlt_ref[...], udrt_ref[...])
    x = x_ref[...]
    shp = x.shape
    acc_a = jnp.broadcast_to(a[0:1], shp)
    acc_b = jnp.broadcast_to(b[0:1], shp)
    acc_c = jnp.broadcast_to(c[0:1], shp)
    acc_d = jnp.broadcast_to(d[0:1], shp)
    acc_w = jnp.zeros(shp, jnp.float32)
    for k in range(1, _NB):
        m = x >= cw[k : k + 1]
        acc_a = jnp.where(m, a[k : k + 1], acc_a)
        acc_b = jnp.where(m, b[k : k + 1], acc_b)
        acc_c = jnp.where(m, c[k : k + 1], acc_c)
        acc_d = jnp.where(m, d[k : k + 1], acc_d)
        acc_w = jnp.where(m, cw[k : k + 1], acc_w)

    sh = x - acc_w
    s2 = sh * sh
    s3 = s2 * sh
    p = acc_a * s3 + acc_b * s2 + acc_c * sh + acc_d
    out_ref[...] = jnp.clip(p, 0.0, 1.0)
    deriv = 3.0 * acc_a * s2 + 2.0 * acc_b * sh + acc_c
    lad = jnp.log(jnp.abs(deriv))
    lad_ref[...] = jnp.sum(lad, axis=1, keepdims=True)


_SC_ROWS = 8192
_BB = 512


def _tc_main(x, uwt, uht, udlt, udrt):
    B, D = x.shape
    nb = _NB
    full = lambda shape: pl.BlockSpec(shape, lambda i: (0, 0))
    out, lad = pl.pallas_call(
        _tc_body,
        grid=(B // _BB,),
        in_specs=[
            pl.BlockSpec((_BB, D), lambda i: (i, 0)),
            full((nb, D)),
            full((nb, D)),
            full((1, D)),
            full((1, D)),
        ],
        out_specs=[
            pl.BlockSpec((_BB, D), lambda i: (i, 0)),
            pl.BlockSpec((_BB, 1), lambda i: (i, 0)),
        ],
        out_shape=[
            jax.ShapeDtypeStruct((B, D), jnp.float32),
            jax.ShapeDtypeStruct((B, 1), jnp.float32),
        ],
    )(x, uwt, uht, udlt, udrt)
    return out, lad.reshape(B)


def _merge_body(dst_ref, src_ref, out_ref):
    out_ref[...] = src_ref[...]


def _merge(full_buf, sc_part):
    B, D = full_buf.shape
    return pl.pallas_call(
        _merge_body,
        grid=(_SC_ROWS // _BB,),
        in_specs=[
            pl.BlockSpec((8, 128), lambda i: (0, 0)),
            pl.BlockSpec((_BB, D), lambda i: (i, 0)),
        ],
        out_specs=pl.BlockSpec((_BB, D), lambda i: (i, 0)),
        out_shape=jax.ShapeDtypeStruct((B, D), jnp.float32),
        input_output_aliases={0: 0},
    )(full_buf, sc_part)


@jax.jit
def kernel(inputs, unnormalized_widths, unnormalized_heights,
           unnorm_derivatives_left, unnorm_derivatives_right):
    B, D = inputs.shape
    cw, a, b, c, d, g = _prep_tables(
        unnormalized_widths, unnormalized_heights,
        unnorm_derivatives_left, unnorm_derivatives_right)
    tab = jnp.concatenate(
        [cw.reshape(-1), a.reshape(-1), b.reshape(-1), c.reshape(-1),
         d.reshape(-1),
         jax.lax.bitcast_convert_type(g, jnp.float32).reshape(-1)])
    sc_out, sc_lad = _sc_main(inputs[:_SC_ROWS].reshape(-1), tab, _SC_ROWS)
    tc_out, tc_lad = _tc_main(
        inputs[_SC_ROWS:], unnormalized_widths.T, unnormalized_heights.T,
        unnorm_derivatives_left.T, unnorm_derivatives_right.T)
    out = jnp.concatenate([sc_out.reshape(_SC_ROWS, D), tc_out], axis=0)
    lad = jnp.concatenate([sc_lad, tc_lad], axis=0)
    return out, lad


# speedup vs baseline: 1.1541x; 1.1541x over previous
import functools

import jax
import jax.numpy as jnp
from jax import lax
from jax.experimental import pallas as pl
from jax.experimental.pallas import tpu as pltpu
from jax.experimental.pallas import tpu_sc as plsc

_MIN_W = 1e-3
_MIN_H = 1e-3
_NB = 32
_NW = 32
_CHUNK = 64


def _softmax1(x):
    m = jnp.max(x, axis=-1, keepdims=True)
    e = jnp.exp(x - m)
    return e / jnp.sum(e, axis=-1, keepdims=True)


def _prep_body(uw_ref, uh_ref, udl_ref, udr_ref,
               cw_ref, a_ref, b_ref, c_ref, d_ref, g_ref):
    nb = _NB
    uw = uw_ref[...]
    uh = uh_ref[...]
    w = _MIN_W + (1.0 - _MIN_W * nb) * _softmax1(uw)
    h = _MIN_H + (1.0 - _MIN_H * nb) * _softmax1(uh)

    row = lax.broadcasted_iota(jnp.int32, (nb, nb), 0)
    col = lax.broadcasted_iota(jnp.int32, (nb, nb), 1)
    tri = (row <= col).astype(jnp.float32)
    cums_w = jnp.dot(w, tri, preferred_element_type=jnp.float32)
    cums_h = jnp.dot(h, tri, preferred_element_type=jnp.float32)

    zero = jnp.zeros_like(w[:, 0:1])
    cw = jnp.concatenate([zero, cums_w[:, : nb - 1]], axis=1)
    dtab = jnp.concatenate([zero, cums_h[:, : nb - 1]], axis=1)

    s = h / w
    s_lo, s_hi = s[:, : nb - 1], s[:, 1:]
    w_lo, w_hi = w[:, : nb - 1], w[:, 1:]
    min1 = jnp.minimum(jnp.abs(s_lo), jnp.abs(s_hi))
    min2 = 0.5 * (w_hi * s_lo + w_lo * s_hi) / (w_lo + w_hi)
    dmid = jnp.minimum(min1, min2) * (jnp.sign(s_lo) + jnp.sign(s_hi))

    d0 = jax.nn.sigmoid(udl_ref[...]) * 3.0 * s[:, 0:1]
    dN = jax.nn.sigmoid(udr_ref[...]) * 3.0 * s[:, nb - 1 : nb]
    dlo = jnp.concatenate([d0, dmid], axis=1)
    dhi = jnp.concatenate([dmid, dN], axis=1)

    cw_ref[...] = jnp.concatenate(
        [cw, jnp.full((cw.shape[0], 1), 2.0, jnp.float32)], axis=1)
    pad = jnp.zeros((cw.shape[0], 1), jnp.float32)
    a_ref[...] = jnp.concatenate([(dlo + dhi - 2.0 * s) / (w * w), pad], 1)
    b_ref[...] = jnp.concatenate([(3.0 * s - 2.0 * dlo - dhi) / w, pad], 1)
    c_ref[...] = jnp.concatenate([dlo, pad], 1)
    d_ref[...] = jnp.concatenate([dtab, pad], 1)

    lane = lax.broadcasted_iota(jnp.int32, (1, 128), 1)
    word = jnp.zeros((cw.shape[0], 128), jnp.int32)
    for p in range(4):
        xg = (4 * lane + p).astype(jnp.float32) * (1.0 / 512.0)
        gp = jnp.zeros((cw.shape[0], 128), jnp.int32)
        for k in range(1, nb):
            gp = gp + jnp.where(xg >= cw[:, k : k + 1], 1, 0)
        word = jnp.bitwise_or(word, jnp.left_shift(gp, 8 * p))
    g_ref[...] = jnp.concatenate(
        [word, jnp.zeros((cw.shape[0], 1), jnp.int32)], 1)


def _prep_tables(uw, uh, udl, udr):
    D, nb = uw.shape
    shape = jax.ShapeDtypeStruct((D, nb), jnp.float32)
    return pl.pallas_call(
        _prep_body,
        out_shape=[jax.ShapeDtypeStruct((D, nb + 1), jnp.float32)] * 5
        + [jax.ShapeDtypeStruct((D, 129), jnp.int32)],
    )(uw, uh, udl, udr)


def _log_abs(y):
    y = jnp.abs(y)
    i = plsc.bitcast(y, jnp.int32)
    e = jnp.right_shift(i, 23) - 127
    m = plsc.bitcast(
        jnp.bitwise_or(jnp.bitwise_and(i, 0x7FFFFF), 0x3F800000), jnp.float32)
    big = m >= 1.4142135381698608
    m = jnp.where(big, m * 0.5, m)
    ef = e.astype(jnp.float32) + jnp.where(big, 1.0, 0.0)
    z = (m - 1.0) / (m + 1.0)
    z2 = z * z
    t = z2 * (1.0 / 7.0) + (1.0 / 5.0)
    t = z2 * t + (1.0 / 3.0)
    t = z2 * t + 1.0
    return ef * 0.6931471805599453 + 2.0 * z * t


def _sc_body(n_rows, x_hbm, cw_h, a_h, b_h, c_h, d_h, g_h, out_hbm, lad_hbm,
             cw_v, a_v, b_v, c_v, d_v, g_v, xin, xout, lad_v):
    wid = lax.axis_index("s") * 2 + lax.axis_index("c")
    pltpu.sync_copy(cw_h, cw_v)
    pltpu.sync_copy(a_h, a_v)
    pltpu.sync_copy(b_h, b_v)
    pltpu.sync_copy(c_h, c_v)
    pltpu.sync_copy(d_h, d_v)
    pltpu.sync_copy(g_h, g_v)

    rows_per_w = n_rows // _NW
    n_chunks = rows_per_w // _CHUNK
    iota = lax.broadcasted_iota(jnp.int32, (16,), 0)

    def chunk_body(ch, carry):
        row0 = wid * rows_per_w + ch * _CHUNK
        pltpu.sync_copy(x_hbm.at[pl.ds(row0 * 256, _CHUNK * 256)], xin)

        @plsc.parallel_loop(0, _CHUNK, unroll=2)
        def row_body(r):
            acc_m = jnp.ones((16,), jnp.float32)
            acc_e = jnp.zeros((16,), jnp.int32)
            xs = [xin[pl.ds(r * 256 + j * 16, 16)] for j in range(16)]
            outs = []
            for j in range(16):
                x = xs[j]
                dvec = iota + (j * 16)
                base33 = dvec * (_NB + 1)
                c512 = (x * 512.0).astype(jnp.int32)
                gw = plsc.load_gather(
                    g_v, [dvec * 129 + jnp.right_shift(c512, 2)])
                sh_amt = jnp.left_shift(jnp.bitwise_and(c512, 3), 3)
                g = jnp.bitwise_and(jnp.right_shift(gw, sh_amt), 0xFF)
                v0 = plsc.load_gather(cw_v, [base33 + g])
                v1 = plsc.load_gather(cw_v, [base33 + g + 1])
                v2 = plsc.load_gather(
                    cw_v, [base33 + jnp.minimum(g + 2, _NB)])
                m1 = x >= v1
                m2 = x >= v2
                binx = (g + jnp.where(m1, 1, 0) + jnp.where(m2, 1, 0))
                wv = jnp.where(m1, jnp.where(m2, v2, v1), v0)
                flat = base33 + binx
                av = plsc.load_gather(a_v, [flat])
                bv = plsc.load_gather(b_v, [flat])
                cv = plsc.load_gather(c_v, [flat])
                dv = plsc.load_gather(d_v, [flat])
                sh = x - wv
                s2 = sh * sh
                s3 = s2 * sh
                p = av * s3 + bv * s2 + cv * sh + dv
                outs.append(jnp.clip(p, 0.0, 1.0))
                deriv = 3.0 * av * s2 + 2.0 * bv * sh + cv
                db = plsc.bitcast(jnp.abs(deriv), jnp.int32)
                acc_e = acc_e + jnp.right_shift(db, 23)
                acc_m = acc_m * plsc.bitcast(
                    jnp.bitwise_or(jnp.bitwise_and(db, 0x7FFFFF), 0x3F800000),
                    jnp.float32)
            for j in range(16):
                xout[pl.ds(r * 256 + j * 16, 16)] = outs[j]
            lad16 = _log_abs(acc_m) + (
                acc_e - 16 * 127).astype(jnp.float32) * 0.6931471805599453
            tot = jnp.broadcast_to(jnp.sum(lad16), (16,))
            plsc.store_scatter(lad_v, [jnp.broadcast_to(r, (16,))], tot,
                               mask=iota == 0)
        pltpu.sync_copy(xout, out_hbm.at[pl.ds(row0 * 256, _CHUNK * 256)])
        pltpu.sync_copy(lad_v, lad_hbm.at[pl.ds(row0, _CHUNK)])
        return carry

    lax.fori_loop(0, n_chunks, chunk_body, 0)


def _sc_main(x_flat, cw, a, b, c, d, g, n_rows):
    B = n_rows
    mesh = plsc.VectorSubcoreMesh(
        core_axis_name="c", subcore_axis_name="s", num_cores=2,
        num_subcores=16)
    f = functools.partial(
        pl.kernel,
        out_type=[
            jax.ShapeDtypeStruct((B * 256,), jnp.float32),
            jax.ShapeDtypeStruct((B,), jnp.float32),
        ],
        mesh=mesh,
        scratch_types=[
            pltpu.VMEM((256 * (_NB + 1),), jnp.float32),
            pltpu.VMEM((256 * (_NB + 1),), jnp.float32),
            pltpu.VMEM((256 * (_NB + 1),), jnp.float32),
            pltpu.VMEM((256 * (_NB + 1),), jnp.float32),
            pltpu.VMEM((256 * (_NB + 1),), jnp.float32),
            pltpu.VMEM((256 * 129,), jnp.int32),
            pltpu.VMEM((_CHUNK * 256,), jnp.float32),
            pltpu.VMEM((_CHUNK * 256,), jnp.float32),
            pltpu.VMEM((_CHUNK,), jnp.float32),
        ],
        compiler_params=pltpu.CompilerParams(needs_layout_passes=False),
    )(functools.partial(_sc_body, n_rows))
    return f(x_flat, cw.reshape(-1), a.reshape(-1), b.reshape(-1),
             c.reshape(-1), d.reshape(-1), g.reshape(-1))


def _softmax0(x):
    m = jnp.max(x, axis=0, keepdims=True)
    e = jnp.exp(x - m)
    return e / jnp.sum(e, axis=0, keepdims=True)


def _tc_tables(uwt, uht, udlt, udrt):
    nb = _NB
    w = _MIN_W + (1.0 - _MIN_W * nb) * _softmax0(uwt)
    h = _MIN_H + (1.0 - _MIN_H * nb) * _softmax0(uht)

    row = lax.broadcasted_iota(jnp.int32, (nb, nb), 0)
    col = lax.broadcasted_iota(jnp.int32, (nb, nb), 1)
    tri = (col <= row).astype(jnp.float32)
    cums_w = jnp.dot(tri, w, preferred_element_type=jnp.float32)
    cums_h = jnp.dot(tri, h, preferred_element_type=jnp.float32)

    zero = jnp.zeros_like(w[0:1])
    cw = jnp.concatenate([zero, cums_w[: nb - 1]], axis=0)
    dtab = jnp.concatenate([zero, cums_h[: nb - 1]], axis=0)

    s = h / w
    s_lo, s_hi = s[: nb - 1], s[1:]
    w_lo, w_hi = w[: nb - 1], w[1:]
    min1 = jnp.minimum(jnp.abs(s_lo), jnp.abs(s_hi))
    min2 = 0.5 * (w_hi * s_lo + w_lo * s_hi) / (w_lo + w_hi)
    dmid = jnp.minimum(min1, min2) * (jnp.sign(s_lo) + jnp.sign(s_hi))

    d0 = jax.nn.sigmoid(udlt) * 3.0 * s[0:1]
    dN = jax.nn.sigmoid(udrt) * 3.0 * s[nb - 1 : nb]
    dlo = jnp.concatenate([d0, dmid], axis=0)
    dhi = jnp.concatenate([dmid, dN], axis=0)

    a = (dlo + dhi - 2.0 * s) / (w * w)
    b = (3.0 * s - 2.0 * dlo - dhi) / w
    return cw, a, b, dlo, dtab


def _tc_body(x_ref, uwt_ref, uht_ref, udlt_ref, udrt_ref, out_ref, lad_ref):
    cw, a, b, c, d = _tc_tables(
        uwt_ref[...], uht_ref[...], udlt_ref[...], udrt_ref[...])
    x = x_ref[...]
    shp = x.shape
    acc_a = jnp.broadcast_to(a[0:1], shp)
    acc_b = jnp.broadcast_to(b[0:1], shp)
    acc_c = jnp.broadcast_to(c[0:1], shp)
    acc_d = jnp.broadcast_to(d[0:1], shp)
    acc_w = jnp.zeros(shp, jnp.float32)
    for k in range(1, _NB):
        m = x >= cw[k : k + 1]
        acc_a = jnp.where(m, a[k : k + 1], acc_a)
        acc_b = jnp.where(m, b[k : k + 1], acc_b)
        acc_c = jnp.where(m, c[k : k + 1], acc_c)
        acc_d = jnp.where(m, d[k : k + 1], acc_d)
        acc_w = jnp.where(m, cw[k : k + 1], acc_w)

    sh = x - acc_w
    s2 = sh * sh
    s3 = s2 * sh
    p = acc_a * s3 + acc_b * s2 + acc_c * sh + acc_d
    out_ref[...] = jnp.clip(p, 0.0, 1.0)
    deriv = 3.0 * acc_a * s2 + 2.0 * acc_b * sh + acc_c
    lad = jnp.log(jnp.abs(deriv))
    lad_ref[...] = jnp.sum(lad, axis=1, keepdims=True)


_SC_ROWS = 8192
_BB = 512


def _tc_main(x, uwt, uht, udlt, udrt):
    B, D = x.shape
    nb = _NB
    full = lambda shape: pl.BlockSpec(shape, lambda i: (0, 0))
    out, lad = pl.pallas_call(
        _tc_body,
        grid=(B // _BB,),
        in_specs=[
            pl.BlockSpec((_BB, D), lambda i: (i, 0)),
            full((nb, D)),
            full((nb, D)),
            full((1, D)),
            full((1, D)),
        ],
        out_specs=[
            pl.BlockSpec((_BB, D), lambda i: (i, 0)),
            pl.BlockSpec((_BB, 1), lambda i: (i, 0)),
        ],
        out_shape=[
            jax.ShapeDtypeStruct((B, D), jnp.float32),
            jax.ShapeDtypeStruct((B, 1), jnp.float32),
        ],
    )(x, uwt, uht, udlt, udrt)
    return out, lad.reshape(B)


def _merge_body(dst_ref, src_ref, out_ref):
    out_ref[...] = src_ref[...]


def _merge(full_buf, sc_part):
    B, D = full_buf.shape
    return pl.pallas_call(
        _merge_body,
        grid=(_SC_ROWS // _BB,),
        in_specs=[
            pl.BlockSpec((8, 128), lambda i: (0, 0)),
            pl.BlockSpec((_BB, D), lambda i: (i, 0)),
        ],
        out_specs=pl.BlockSpec((_BB, D), lambda i: (i, 0)),
        out_shape=jax.ShapeDtypeStruct((B, D), jnp.float32),
        input_output_aliases={0: 0},
    )(full_buf, sc_part)


@jax.jit
def kernel(inputs, unnormalized_widths, unnormalized_heights,
           unnorm_derivatives_left, unnorm_derivatives_right):
    B, D = inputs.shape
    cw, a, b, c, d, g = _prep_tables(
        unnormalized_widths, unnormalized_heights,
        unnorm_derivatives_left, unnorm_derivatives_right)
    sc_out, sc_lad = _sc_main(
        inputs[:_SC_ROWS].reshape(-1), cw, a, b, c, d, g, _SC_ROWS)
    tc_out, tc_lad = _tc_main(
        inputs[_SC_ROWS:], unnormalized_widths.T, unnormalized_heights.T,
        unnorm_derivatives_left.T, unnorm_derivatives_right.T)
    out = jnp.concatenate([sc_out.reshape(_SC_ROWS, D), tc_out], axis=0)
    lad = jnp.concatenate([sc_lad, tc_lad], axis=0)
    return out, lad


# speedup vs baseline: 1.1584x; 1.0038x over previous
import functools

import jax
import jax.numpy as jnp
from jax import lax
from jax.experimental import pallas as pl
from jax.experimental.pallas import tpu as pltpu
from jax.experimental.pallas import tpu_sc as plsc

_MIN_W = 1e-3
_MIN_H = 1e-3
_NB = 32
_NW = 32
_CHUNK = 64


def _softmax1(x):
    m = jnp.max(x, axis=-1, keepdims=True)
    e = jnp.exp(x - m)
    return e / jnp.sum(e, axis=-1, keepdims=True)


def _prep_body(uw_ref, uh_ref, udl_ref, udr_ref,
               cw_ref, a_ref, b_ref, c_ref, d_ref, g_ref):
    nb = _NB
    uw = uw_ref[...]
    uh = uh_ref[...]
    w = _MIN_W + (1.0 - _MIN_W * nb) * _softmax1(uw)
    h = _MIN_H + (1.0 - _MIN_H * nb) * _softmax1(uh)

    row = lax.broadcasted_iota(jnp.int32, (nb, nb), 0)
    col = lax.broadcasted_iota(jnp.int32, (nb, nb), 1)
    tri = (row <= col).astype(jnp.float32)
    cums_w = jnp.dot(w, tri, preferred_element_type=jnp.float32)
    cums_h = jnp.dot(h, tri, preferred_element_type=jnp.float32)

    zero = jnp.zeros_like(w[:, 0:1])
    cw = jnp.concatenate([zero, cums_w[:, : nb - 1]], axis=1)
    dtab = jnp.concatenate([zero, cums_h[:, : nb - 1]], axis=1)

    s = h / w
    s_lo, s_hi = s[:, : nb - 1], s[:, 1:]
    w_lo, w_hi = w[:, : nb - 1], w[:, 1:]
    min1 = jnp.minimum(jnp.abs(s_lo), jnp.abs(s_hi))
    min2 = 0.5 * (w_hi * s_lo + w_lo * s_hi) / (w_lo + w_hi)
    dmid = jnp.minimum(min1, min2) * (jnp.sign(s_lo) + jnp.sign(s_hi))

    d0 = jax.nn.sigmoid(udl_ref[...]) * 3.0 * s[:, 0:1]
    dN = jax.nn.sigmoid(udr_ref[...]) * 3.0 * s[:, nb - 1 : nb]
    dlo = jnp.concatenate([d0, dmid], axis=1)
    dhi = jnp.concatenate([dmid, dN], axis=1)

    cw_ref[...] = jnp.concatenate(
        [cw, jnp.full((cw.shape[0], 1), 2.0, jnp.float32)], axis=1)
    pad = jnp.zeros((cw.shape[0], 1), jnp.float32)
    a_ref[...] = jnp.concatenate([(dlo + dhi - 2.0 * s) / (w * w), pad], 1)
    b_ref[...] = jnp.concatenate([(3.0 * s - 2.0 * dlo - dhi) / w, pad], 1)
    c_ref[...] = jnp.concatenate([dlo, pad], 1)
    d_ref[...] = jnp.concatenate([dtab, pad], 1)

    lane = lax.broadcasted_iota(jnp.int32, (1, 128), 1)
    word = jnp.zeros((cw.shape[0], 128), jnp.int32)
    for p in range(4):
        xg = (4 * lane + p).astype(jnp.float32) * (1.0 / 512.0)
        gp = jnp.zeros((cw.shape[0], 128), jnp.int32)
        for k in range(1, nb):
            gp = gp + jnp.where(xg >= cw[:, k : k + 1], 1, 0)
        word = jnp.bitwise_or(word, jnp.left_shift(gp, 8 * p))
    g_ref[...] = jnp.concatenate(
        [word, jnp.zeros((cw.shape[0], 1), jnp.int32)], 1)


def _prep_tables(uw, uh, udl, udr):
    D, nb = uw.shape
    shape = jax.ShapeDtypeStruct((D, nb), jnp.float32)
    return pl.pallas_call(
        _prep_body,
        out_shape=[jax.ShapeDtypeStruct((D, nb + 1), jnp.float32)] * 5
        + [jax.ShapeDtypeStruct((D, 129), jnp.int32)],
    )(uw, uh, udl, udr)


def _log_abs(y):
    y = jnp.abs(y)
    i = plsc.bitcast(y, jnp.int32)
    e = jnp.right_shift(i, 23) - 127
    m = plsc.bitcast(
        jnp.bitwise_or(jnp.bitwise_and(i, 0x7FFFFF), 0x3F800000), jnp.float32)
    big = m >= 1.4142135381698608
    m = jnp.where(big, m * 0.5, m)
    ef = e.astype(jnp.float32) + jnp.where(big, 1.0, 0.0)
    z = (m - 1.0) / (m + 1.0)
    z2 = z * z
    t = z2 * (1.0 / 7.0) + (1.0 / 5.0)
    t = z2 * t + (1.0 / 3.0)
    t = z2 * t + 1.0
    return ef * 0.6931471805599453 + 2.0 * z * t


def _sc_body(n_rows, x_hbm, cw_h, a_h, b_h, c_h, d_h, g_h, out_hbm, lad_hbm,
             cw_v, a_v, b_v, c_v, d_v, g_v, xin, xout, lad_v):
    wid = lax.axis_index("s") * 2 + lax.axis_index("c")
    pltpu.sync_copy(cw_h, cw_v)
    pltpu.sync_copy(a_h, a_v)
    pltpu.sync_copy(b_h, b_v)
    pltpu.sync_copy(c_h, c_v)
    pltpu.sync_copy(d_h, d_v)
    pltpu.sync_copy(g_h, g_v)

    rows_per_w = n_rows // _NW
    n_chunks = rows_per_w // _CHUNK
    iota = lax.broadcasted_iota(jnp.int32, (16,), 0)

    def chunk_body(ch, carry):
        row0 = wid * rows_per_w + ch * _CHUNK
        pltpu.sync_copy(x_hbm.at[pl.ds(row0 * 256, _CHUNK * 256)], xin)

        @plsc.parallel_loop(0, _CHUNK, unroll=4)
        def row_body(r):
            acc_m = jnp.ones((16,), jnp.float32)
            acc_e = jnp.zeros((16,), jnp.int32)
            xs = [xin[pl.ds(r * 256 + j * 16, 16)] for j in range(16)]
            outs = []
            for j in range(16):
                x = xs[j]
                dvec = iota + (j * 16)
                base33 = dvec * (_NB + 1)
                c512 = (x * 512.0).astype(jnp.int32)
                gw = plsc.load_gather(
                    g_v, [dvec * 129 + jnp.right_shift(c512, 2)])
                sh_amt = jnp.left_shift(jnp.bitwise_and(c512, 3), 3)
                g = jnp.bitwise_and(jnp.right_shift(gw, sh_amt), 0xFF)
                v0 = plsc.load_gather(cw_v, [base33 + g])
                v1 = plsc.load_gather(cw_v, [base33 + g + 1])
                v2 = plsc.load_gather(
                    cw_v, [base33 + jnp.minimum(g + 2, _NB)])
                m1 = x >= v1
                m2 = x >= v2
                binx = (g + jnp.where(m1, 1, 0) + jnp.where(m2, 1, 0))
                wv = jnp.where(m1, jnp.where(m2, v2, v1), v0)
                flat = base33 + binx
                av = plsc.load_gather(a_v, [flat])
                bv = plsc.load_gather(b_v, [flat])
                cv = plsc.load_gather(c_v, [flat])
                dv = plsc.load_gather(d_v, [flat])
                sh = x - wv
                s2 = sh * sh
                s3 = s2 * sh
                p = av * s3 + bv * s2 + cv * sh + dv
                outs.append(jnp.clip(p, 0.0, 1.0))
                deriv = 3.0 * av * s2 + 2.0 * bv * sh + cv
                db = plsc.bitcast(jnp.abs(deriv), jnp.int32)
                acc_e = acc_e + jnp.right_shift(db, 23)
                acc_m = acc_m * plsc.bitcast(
                    jnp.bitwise_or(jnp.bitwise_and(db, 0x7FFFFF), 0x3F800000),
                    jnp.float32)
            for j in range(16):
                xout[pl.ds(r * 256 + j * 16, 16)] = outs[j]
            lad16 = _log_abs(acc_m) + (
                acc_e - 16 * 127).astype(jnp.float32) * 0.6931471805599453
            tot = jnp.broadcast_to(jnp.sum(lad16), (16,))
            plsc.store_scatter(lad_v, [jnp.broadcast_to(r, (16,))], tot,
                               mask=iota == 0)
        pltpu.sync_copy(xout, out_hbm.at[pl.ds(row0 * 256, _CHUNK * 256)])
        pltpu.sync_copy(lad_v, lad_hbm.at[pl.ds(row0, _CHUNK)])
        return carry

    lax.fori_loop(0, n_chunks, chunk_body, 0)


def _sc_main(x_flat, cw, a, b, c, d, g, n_rows):
    B = n_rows
    mesh = plsc.VectorSubcoreMesh(
        core_axis_name="c", subcore_axis_name="s", num_cores=2,
        num_subcores=16)
    f = functools.partial(
        pl.kernel,
        out_type=[
            jax.ShapeDtypeStruct((B * 256,), jnp.float32),
            jax.ShapeDtypeStruct((B,), jnp.float32),
        ],
        mesh=mesh,
        scratch_types=[
            pltpu.VMEM((256 * (_NB + 1),), jnp.float32),
            pltpu.VMEM((256 * (_NB + 1),), jnp.float32),
            pltpu.VMEM((256 * (_NB + 1),), jnp.float32),
            pltpu.VMEM((256 * (_NB + 1),), jnp.float32),
            pltpu.VMEM((256 * (_NB + 1),), jnp.float32),
            pltpu.VMEM((256 * 129,), jnp.int32),
            pltpu.VMEM((_CHUNK * 256,), jnp.float32),
            pltpu.VMEM((_CHUNK * 256,), jnp.float32),
            pltpu.VMEM((_CHUNK,), jnp.float32),
        ],
        compiler_params=pltpu.CompilerParams(needs_layout_passes=False),
    )(functools.partial(_sc_body, n_rows))
    return f(x_flat, cw.reshape(-1), a.reshape(-1), b.reshape(-1),
             c.reshape(-1), d.reshape(-1), g.reshape(-1))


def _softmax0(x):
    m = jnp.max(x, axis=0, keepdims=True)
    e = jnp.exp(x - m)
    return e / jnp.sum(e, axis=0, keepdims=True)


def _tc_tables(uwt, uht, udlt, udrt):
    nb = _NB
    w = _MIN_W + (1.0 - _MIN_W * nb) * _softmax0(uwt)
    h = _MIN_H + (1.0 - _MIN_H * nb) * _softmax0(uht)

    row = lax.broadcasted_iota(jnp.int32, (nb, nb), 0)
    col = lax.broadcasted_iota(jnp.int32, (nb, nb), 1)
    tri = (col <= row).astype(jnp.float32)
    cums_w = jnp.dot(tri, w, preferred_element_type=jnp.float32)
    cums_h = jnp.dot(tri, h, preferred_element_type=jnp.float32)

    zero = jnp.zeros_like(w[0:1])
    cw = jnp.concatenate([zero, cums_w[: nb - 1]], axis=0)
    dtab = jnp.concatenate([zero, cums_h[: nb - 1]], axis=0)

    s = h / w
    s_lo, s_hi = s[: nb - 1], s[1:]
    w_lo, w_hi = w[: nb - 1], w[1:]
    min1 = jnp.minimum(jnp.abs(s_lo), jnp.abs(s_hi))
    min2 = 0.5 * (w_hi * s_lo + w_lo * s_hi) / (w_lo + w_hi)
    dmid = jnp.minimum(min1, min2) * (jnp.sign(s_lo) + jnp.sign(s_hi))

    d0 = jax.nn.sigmoid(udlt) * 3.0 * s[0:1]
    dN = jax.nn.sigmoid(udrt) * 3.0 * s[nb - 1 : nb]
    dlo = jnp.concatenate([d0, dmid], axis=0)
    dhi = jnp.concatenate([dmid, dN], axis=0)

    a = (dlo + dhi - 2.0 * s) / (w * w)
    b = (3.0 * s - 2.0 * dlo - dhi) / w
    return cw, a, b, dlo, dtab


def _tc_body(x_ref, uwt_ref, uht_ref, udlt_ref, udrt_ref, out_ref, lad_ref):
    cw, a, b, c, d = _tc_tables(
        uwt_ref[...], uht_ref[...], udlt_ref[...], udrt_ref[...])
    x = x_ref[...]
    shp = x.shape
    acc_a = jnp.broadcast_to(a[0:1], shp)
    acc_b = jnp.broadcast_to(b[0:1], shp)
    acc_c = jnp.broadcast_to(c[0:1], shp)
    acc_d = jnp.broadcast_to(d[0:1], shp)
    acc_w = jnp.zeros(shp, jnp.float32)
    for k in range(1, _NB):
        m = x >= cw[k : k + 1]
        acc_a = jnp.where(m, a[k : k + 1], acc_a)
        acc_b = jnp.where(m, b[k : k + 1], acc_b)
        acc_c = jnp.where(m, c[k : k + 1], acc_c)
        acc_d = jnp.where(m, d[k : k + 1], acc_d)
        acc_w = jnp.where(m, cw[k : k + 1], acc_w)

    sh = x - acc_w
    s2 = sh * sh
    s3 = s2 * sh
    p = acc_a * s3 + acc_b * s2 + acc_c * sh + acc_d
    out_ref[...] = jnp.clip(p, 0.0, 1.0)
    deriv = 3.0 * acc_a * s2 + 2.0 * acc_b * sh + acc_c
    lad = jnp.log(jnp.abs(deriv))
    lad_ref[...] = jnp.sum(lad, axis=1, keepdims=True)


_SC_ROWS = 8192
_BB = 512


def _tc_main(x, uwt, uht, udlt, udrt):
    B, D = x.shape
    nb = _NB
    full = lambda shape: pl.BlockSpec(shape, lambda i: (0, 0))
    out, lad = pl.pallas_call(
        _tc_body,
        grid=(B // _BB,),
        in_specs=[
            pl.BlockSpec((_BB, D), lambda i: (i, 0)),
            full((nb, D)),
            full((nb, D)),
            full((1, D)),
            full((1, D)),
        ],
        out_specs=[
            pl.BlockSpec((_BB, D), lambda i: (i, 0)),
            pl.BlockSpec((_BB, 1), lambda i: (i, 0)),
        ],
        out_shape=[
            jax.ShapeDtypeStruct((B, D), jnp.float32),
            jax.ShapeDtypeStruct((B, 1), jnp.float32),
        ],
    )(x, uwt, uht, udlt, udrt)
    return out, lad.reshape(B)


def _merge_body(dst_ref, src_ref, out_ref):
    out_ref[...] = src_ref[...]


def _merge(full_buf, sc_part):
    B, D = full_buf.shape
    return pl.pallas_call(
        _merge_body,
        grid=(_SC_ROWS // _BB,),
        in_specs=[
            pl.BlockSpec((8, 128), lambda i: (0, 0)),
            pl.BlockSpec((_BB, D), lambda i: (i, 0)),
        ],
        out_specs=pl.BlockSpec((_BB, D), lambda i: (i, 0)),
        out_shape=jax.ShapeDtypeStruct((B, D), jnp.float32),
        input_output_aliases={0: 0},
    )(full_buf, sc_part)


@jax.jit
def kernel(inputs, unnormalized_widths, unnormalized_heights,
           unnorm_derivatives_left, unnorm_derivatives_right):
    B, D = inputs.shape
    cw, a, b, c, d, g = _prep_tables(
        unnormalized_widths, unnormalized_heights,
        unnorm_derivatives_left, unnorm_derivatives_right)
    sc_out, sc_lad = _sc_main(
        inputs[:_SC_ROWS].reshape(-1), cw, a, b, c, d, g, _SC_ROWS)
    tc_out, tc_lad = _tc_main(
        inputs[_SC_ROWS:], unnormalized_widths.T, unnormalized_heights.T,
        unnorm_derivatives_left.T, unnorm_derivatives_right.T)
    out = jnp.concatenate([sc_out.reshape(_SC_ROWS, D), tc_out], axis=0)
    lad = jnp.concatenate([sc_lad, tc_lad], axis=0)
    return out, lad
